# Initial kernel scaffold; baseline (speedup 1.0000x reference)
import jax
import jax.numpy as jnp
from jax.experimental import pallas as pl


def kernel(inputs_embeds, images_seq_mask, stacked_image_feats):
    raise NotImplementedError("write your pallas kernel here")


# speedup vs baseline: 1548.1164x; 1548.1164x over previous
import functools

import jax
import jax.numpy as jnp
from jax import lax
from jax.experimental import pallas as pl
from jax.experimental.pallas import tpu as pltpu
from jax.experimental.pallas import tpu_sc as plsc

B, S, D = 4, 4096, 1024
T = B * S
N_IMG = 8192
MAXI = N_IMG * D - 1
NC, NS, L = 2, 16, 16
NW = NC * NS
TPW = T // NW
K = 32
NCHUNK = TPW // K
NV = TPW // L


def _body(mask_hbm, feats_hbm, feats16_hbm, out_hbm,
          mask_v, c_v, gidx_v, sidx_v, rowbuf, chunkbuf,
          sem_g, sem_c, sem_o):
  cid = lax.axis_index("c")
  sid = lax.axis_index("s")
  wid = sid * NC + cid
  t0 = wid * TPW

  pltpu.sync_copy(mask_hbm, mask_v)

  def _red(j, acc):
    return acc + mask_v[pl.ds(j * L, L)]
  accv = lax.fori_loop(0, wid * NV, _red, jnp.zeros((L,), jnp.int32))
  base0 = jnp.sum(accv)

  def _scan(j, carry):
    v = mask_v[pl.ds(t0 + j * L, L)]
    incl = jnp.cumsum(v)
    c_v[pl.ds(j * L, L)] = carry + incl - v
    return carry + jnp.sum(v)
  lax.fori_loop(0, NV, _scan, base0)

  def _chunk(i, _):
    tok0 = i * K
    flags = []
    lanes = []
    for h in range(K // L):
      c = c_v[pl.ds(tok0 + h * L, L)]
      m = mask_v[pl.ds(t0 + tok0 + h * L, L)]
      is_row = (m > 0) & (c < N_IMG)
      gidx_v[pl.ds(h * L, L)] = jnp.where(is_row, c, 0)
      e = jnp.clip(c * D - (1 - m), 0, MAXI)
      sidx_v[pl.ds(h * L, L)] = e >> 7
      flags.append(1 - is_row.astype(jnp.int32))
      lanes.append(e & 127)

    pltpu.async_copy(feats_hbm.at[gidx_v], rowbuf, sem_g).wait()
    pltpu.async_copy(feats16_hbm.at[sidx_v], chunkbuf, sem_c).wait()

    for h in range(K // L):
      splats = plsc.load_gather(
          chunkbuf, [h * L + lax.iota(jnp.int32, L), lanes[h]])
      for k in range(L):
        @pl.when(flags[h][k] != 0)
        def _(h=h, k=k, splats=splats):
          sv = jnp.full((L,), splats[k])

          def _fill_j(j, _):
            for u in range(8):
              rowbuf[h * L + k, pl.ds((j * 8 + u) * L, L)] = sv
            return 0
          lax.fori_loop(0, D // (8 * L), _fill_j, 0)

    pltpu.async_copy(rowbuf, out_hbm.at[pl.ds(t0 + tok0, K)], sem_o).wait()
    return 0
  lax.fori_loop(0, NCHUNK, _chunk, 0)


_mesh = plsc.VectorSubcoreMesh(core_axis_name="c", subcore_axis_name="s")

_sc_call = functools.partial(
    pl.kernel,
    out_type=jax.ShapeDtypeStruct((T, D), jnp.float32),
    mesh=_mesh,
    compiler_params=pltpu.CompilerParams(needs_layout_passes=False),
    scratch_types=[
        pltpu.VMEM((T,), jnp.int32),
        pltpu.VMEM((TPW,), jnp.int32),
        pltpu.VMEM((K,), jnp.int32),
        pltpu.VMEM((K,), jnp.int32),
        pltpu.VMEM((K, D), jnp.float32),
        pltpu.VMEM((K, 128), jnp.float32),
        pltpu.SemaphoreType.DMA,
        pltpu.SemaphoreType.DMA,
        pltpu.SemaphoreType.DMA,
    ],
)(_body)


@jax.jit
def kernel(inputs_embeds, images_seq_mask, stacked_image_feats):
  del inputs_embeds
  mask_i = images_seq_mask.reshape(-1).astype(jnp.int32)
  feats16 = stacked_image_feats.reshape(-1, 128)
  out = _sc_call(mask_i, stacked_image_feats, feats16)
  return out.reshape(-1)


# speedup vs baseline: 1686.5001x; 1.0894x over previous
import functools

import jax
import jax.numpy as jnp
from jax import lax
from jax.experimental import pallas as pl
from jax.experimental.pallas import tpu as pltpu
from jax.experimental.pallas import tpu_sc as plsc

B, S, D = 4, 4096, 1024
T = B * S
N_IMG = 8192
MAXI = N_IMG * D - 1
NC, NS, L = 2, 16, 16
NW = NC * NS
TPW = T // NW
K = 32
NCHUNK = TPW // K
NPAIR = NCHUNK // 2
NV = TPW // L
R = D // 128
RPC = K * R


def _body(mask_hbm, src_hbm, out_hbm,
          mask_v, crow_v, gidx_v, sidx_v, lane_v, flag_v,
          rb0, rb1, cb0, cb1,
          sem_g0, sem_g1, sem_s0, sem_s1):
  cid = lax.axis_index("c")
  sid = lax.axis_index("s")
  wid = sid * NC + cid
  t0 = wid * TPW

  pltpu.sync_copy(mask_hbm, mask_v)

  def _red(j, acc):
    return acc + mask_v[pl.ds(j * L, L)]
  accv = lax.fori_loop(0, wid * NV, _red, jnp.zeros((L,), jnp.int32))
  base0 = jnp.sum(accv)

  iota = lax.iota(jnp.int32, L)

  def _scan(j, carry):
    v = mask_v[pl.ds(t0 + j * L, L)]
    incl = jnp.cumsum(v)
    c = carry + incl - v
    is_row = (v > 0) & (c < N_IMG)
    crow_v[pl.ds(j * L, L)] = jnp.where(is_row, c, 0)
    e = jnp.clip(c * D - (1 - v), 0, MAXI)
    sidx_v[pl.ds(j * L, L)] = e >> 7
    lane_v[pl.ds(j * L, L)] = e & 127
    flag_v[pl.ds(j * L, L)] = 1 - is_row.astype(jnp.int32)
    return carry + jnp.sum(v)
  lax.fori_loop(0, NV, _scan, base0)

  def _g8(v16, _):
    tokv = 2 * v16 + (iota >> 3)
    cr = plsc.load_gather(crow_v, [tokv])
    gidx_v[pl.ds(v16 * L, L)] = cr * R + (iota & 7)
    return 0
  lax.fori_loop(0, TPW * R // L, _g8, 0)

  def issue_gather(i, rb, cb, sem):
    pltpu.async_copy(src_hbm.at[gidx_v.at[pl.ds(i * RPC, 128)]],
                     rb.at[pl.ds(0, 128)], sem)
    pltpu.async_copy(src_hbm.at[gidx_v.at[pl.ds(i * RPC + 128, 128)]],
                     rb.at[pl.ds(128, 128)], sem)
    pltpu.async_copy(src_hbm.at[sidx_v.at[pl.ds(i * K, K)]], cb, sem)

  def drain_gather(rb, cb, sem):
    pltpu.make_async_copy(src_hbm.at[pl.ds(0, 128)],
                          rb.at[pl.ds(0, 128)], sem).wait()
    pltpu.make_async_copy(src_hbm.at[pl.ds(0, 128)],
                          rb.at[pl.ds(128, 128)], sem).wait()
    pltpu.make_async_copy(src_hbm.at[pl.ds(0, K)], cb, sem).wait()

  def issue_store(i, rb, sem):
    pltpu.async_copy(rb, out_hbm.at[pl.ds((t0 + i * K) * R, RPC)], sem)

  def drain_store(rb, sem):
    pltpu.make_async_copy(rb, out_hbm.at[pl.ds(0, RPC)], sem).wait()

  def fill(i, rb, cb):
    for h in range(K // L):
      off = i * K + h * L
      fvec = flag_v[pl.ds(off, L)]
      lvec = lane_v[pl.ds(off, L)]
      splats = plsc.load_gather(cb, [h * L + iota, lvec])
      for k in range(L):
        @pl.when(fvec[k] != 0)
        def _(h=h, k=k, splats=splats):
          sv = jnp.full((L,), splats[k])
          row0 = (h * L + k) * R

          def _fill_j(j, _):
            for u in range(8):
              rb[row0 + j, pl.ds(u * L, L)] = sv
            return 0
          lax.fori_loop(0, R, _fill_j, 0)

  issue_gather(0, rb0, cb0, sem_g0)

  def _pair(g, _):
    a = 2 * g
    b = a + 1

    @pl.when(g > 0)
    def _():
      drain_store(rb1, sem_s1)
    issue_gather(b, rb1, cb1, sem_g1)

    drain_gather(rb0, cb0, sem_g0)
    fill(a, rb0, cb0)
    issue_store(a, rb0, sem_s0)

    drain_gather(rb1, cb1, sem_g1)
    fill(b, rb1, cb1)
    issue_store(b, rb1, sem_s1)

    @pl.when(g < NPAIR - 1)
    def _():
      drain_store(rb0, sem_s0)
      issue_gather(a + 2, rb0, cb0, sem_g0)
    return 0
  lax.fori_loop(0, NPAIR, _pair, 0)

  drain_store(rb0, sem_s0)
  drain_store(rb1, sem_s1)


_mesh = plsc.VectorSubcoreMesh(core_axis_name="c", subcore_axis_name="s")

_sc_call = functools.partial(
    pl.kernel,
    out_type=jax.ShapeDtypeStruct((T * R, 128), jnp.float32),
    mesh=_mesh,
    compiler_params=pltpu.CompilerParams(needs_layout_passes=False),
    scratch_types=[
        pltpu.VMEM((T,), jnp.int32),
        pltpu.VMEM((TPW,), jnp.int32),
        pltpu.VMEM((TPW * R,), jnp.int32),
        pltpu.VMEM((TPW,), jnp.int32),
        pltpu.VMEM((TPW,), jnp.int32),
        pltpu.VMEM((TPW,), jnp.int32),
        pltpu.VMEM((RPC, 128), jnp.float32),
        pltpu.VMEM((RPC, 128), jnp.float32),
        pltpu.VMEM((K, 128), jnp.float32),
        pltpu.VMEM((K, 128), jnp.float32),
        pltpu.SemaphoreType.DMA,
        pltpu.SemaphoreType.DMA,
        pltpu.SemaphoreType.DMA,
        pltpu.SemaphoreType.DMA,
    ],
)(_body)


@jax.jit
def kernel(inputs_embeds, images_seq_mask, stacked_image_feats):
  del inputs_embeds
  mask_i = images_seq_mask.reshape(-1).astype(jnp.int32)
  src128 = stacked_image_feats.reshape(-1, 128)
  out = _sc_call(mask_i, src128)
  return out.reshape(-1)


# speedup vs baseline: 4810.5734x; 2.8524x over previous
import functools

import jax
import jax.numpy as jnp
from jax import lax
from jax.experimental import pallas as pl
from jax.experimental.pallas import tpu as pltpu
from jax.experimental.pallas import tpu_sc as plsc

B, S, D = 4, 4096, 1024
T = B * S
N_IMG = 8192
MAXI = N_IMG * D - 1
NC, NS, L = 2, 16, 16
NW = NC * NS
TPW = T // NW
K = 32
NCHUNK = TPW // K
NPAIR = NCHUNK // 2
NV = TPW // L


def _body(mask_hbm, src_hbm, out_hbm,
          mask_v, grow_v, lane_v, flag_v,
          rb0, rb1,
          sem_g0, sem_g1, sem_s0, sem_s1):
  cid = lax.axis_index("c")
  sid = lax.axis_index("s")
  wid = sid * NC + cid
  t0 = wid * TPW

  pltpu.sync_copy(mask_hbm, mask_v)

  def _red(j, acc):
    return acc + mask_v[pl.ds(j * L, L)]
  accv = lax.fori_loop(0, wid * NV, _red, jnp.zeros((L,), jnp.int32))
  base0 = jnp.sum(accv)

  iota = lax.iota(jnp.int32, L)

  def _scan(j, carry):
    v = mask_v[pl.ds(t0 + j * L, L)]
    incl = jnp.cumsum(v)
    c = carry + incl - v
    is_row = (v > 0) & (c < N_IMG)
    e = jnp.clip(c * D - (1 - v), 0, MAXI)
    grow_v[pl.ds(j * L, L)] = e >> 10
    lane_v[pl.ds(j * L, L)] = e & (D - 1)
    flag_v[pl.ds(j * L, L)] = 1 - is_row.astype(jnp.int32)
    return carry + jnp.sum(v)
  lax.fori_loop(0, NV, _scan, base0)

  def issue_gather(i, rb, sem):
    pltpu.async_copy(src_hbm.at[grow_v.at[pl.ds(i * K, K)]], rb, sem)

  def drain_gather(rb, sem):
    pltpu.make_async_copy(src_hbm.at[pl.ds(0, K)], rb, sem).wait()

  def issue_store(i, rb, sem):
    pltpu.async_copy(rb, out_hbm.at[pl.ds(t0 + i * K, K)], sem)

  def drain_store(rb, sem):
    pltpu.make_async_copy(rb, out_hbm.at[pl.ds(0, K)], sem).wait()

  def fill(i, rb):
    for h in range(K // L):
      off = i * K + h * L
      fvec = flag_v[pl.ds(off, L)]
      lvec = lane_v[pl.ds(off, L)]
      splats = plsc.load_gather(rb, [h * L + iota, lvec])
      for k in range(L):
        @pl.when(fvec[k] != 0)
        def _(h=h, k=k, splats=splats):
          sv = jnp.full((L,), splats[k])
          row = h * L + k

          def _fill_j(j, _):
            for u in range(8):
              rb[row, pl.ds((j * 8 + u) * L, L)] = sv
            return 0
          lax.fori_loop(0, D // (8 * L), _fill_j, 0)

  issue_gather(0, rb0, sem_g0)

  def _pair(g, _):
    a = 2 * g
    b = a + 1

    @pl.when(g > 0)
    def _():
      drain_store(rb1, sem_s1)
    issue_gather(b, rb1, sem_g1)

    drain_gather(rb0, sem_g0)
    fill(a, rb0)
    issue_store(a, rb0, sem_s0)

    drain_gather(rb1, sem_g1)
    fill(b, rb1)
    issue_store(b, rb1, sem_s1)

    @pl.when(g < NPAIR - 1)
    def _():
      drain_store(rb0, sem_s0)
      issue_gather(a + 2, rb0, sem_g0)
    return 0
  lax.fori_loop(0, NPAIR, _pair, 0)

  drain_store(rb0, sem_s0)
  drain_store(rb1, sem_s1)


_mesh = plsc.VectorSubcoreMesh(core_axis_name="c", subcore_axis_name="s")

_sc_call = functools.partial(
    pl.kernel,
    out_type=jax.ShapeDtypeStruct((T, D), jnp.float32),
    mesh=_mesh,
    compiler_params=pltpu.CompilerParams(needs_layout_passes=False),
    scratch_types=[
        pltpu.VMEM((T,), jnp.int32),
        pltpu.VMEM((TPW,), jnp.int32),
        pltpu.VMEM((TPW,), jnp.int32),
        pltpu.VMEM((TPW,), jnp.int32),
        pltpu.VMEM((K, D), jnp.float32),
        pltpu.VMEM((K, D), jnp.float32),
        pltpu.SemaphoreType.DMA,
        pltpu.SemaphoreType.DMA,
        pltpu.SemaphoreType.DMA,
        pltpu.SemaphoreType.DMA,
    ],
)(_body)


@jax.jit
def kernel(inputs_embeds, images_seq_mask, stacked_image_feats):
  del inputs_embeds
  mask_i = images_seq_mask.reshape(-1).astype(jnp.int32)
  out = _sc_call(mask_i, stacked_image_feats)
  return out.reshape(-1)


# speedup vs baseline: 5767.5366x; 1.1989x over previous
import functools

import jax
import jax.numpy as jnp
from jax import lax
from jax.experimental import pallas as pl
from jax.experimental.pallas import tpu as pltpu
from jax.experimental.pallas import tpu_sc as plsc

B, S, D = 4, 4096, 1024
T = B * S
N_IMG = 8192
MAXI = N_IMG * D - 1
NC, NS, L = 2, 16, 16
NW = NC * NS
TPW = T // NW
K = 16
NCHUNK = TPW // K
NSLOT = 4
NV = TPW // L


def _body(mask_hbm, src_hbm, out_hbm,
          mask_v, grow_v, lane_v, flag_v,
          rb0, rb1, rb2, rb3,
          sem_g0, sem_g1, sem_g2, sem_g3,
          sem_s0, sem_s1, sem_s2, sem_s3):
  cid = lax.axis_index("c")
  sid = lax.axis_index("s")
  wid = sid * NC + cid
  t0 = wid * TPW

  pltpu.sync_copy(mask_hbm, mask_v)

  def _red(j, acc):
    a0, a1 = acc
    for u in range(0, 8, 2):
      a0 = a0 + mask_v[pl.ds((j * 8 + u) * L, L)]
      a1 = a1 + mask_v[pl.ds((j * 8 + u + 1) * L, L)]
    return (a0, a1)
  z = jnp.zeros((L,), jnp.int32)
  acc0, acc1 = lax.fori_loop(0, wid * (NV // 8), _red, (z, z))
  base0 = jnp.sum(acc0 + acc1)

  iota = lax.iota(jnp.int32, L)

  def _scan(j, carry):
    v = mask_v[pl.ds(t0 + j * L, L)]
    incl = jnp.cumsum(v)
    c = carry + incl - v
    is_row = (v > 0) & (c < N_IMG)
    e = jnp.clip(c * D - (1 - v), 0, MAXI)
    grow_v[pl.ds(j * L, L)] = e >> 10
    lane_v[pl.ds(j * L, L)] = e & (D - 1)
    flag_v[pl.ds(j * L, L)] = 1 - is_row.astype(jnp.int32)
    return carry + jnp.sum(v)
  lax.fori_loop(0, NV, _scan, base0)

  def issue_gather(i, rb, sem):
    pltpu.async_copy(src_hbm.at[grow_v.at[pl.ds(i * K, K)]], rb, sem)

  def drain_gather(rb, sem):
    pltpu.make_async_copy(src_hbm.at[pl.ds(0, K)], rb, sem).wait()

  def issue_store(i, rb, sem):
    pltpu.async_copy(rb, out_hbm.at[pl.ds(t0 + i * K, K)], sem)

  def drain_store(rb, sem):
    pltpu.make_async_copy(rb, out_hbm.at[pl.ds(0, K)], sem).wait()

  def fill(i, rb):
    for h in range(K // L):
      off = i * K + h * L
      fvec = flag_v[pl.ds(off, L)]
      lvec = lane_v[pl.ds(off, L)]
      splats = plsc.load_gather(rb, [h * L + iota, lvec >> 7, lvec & 127])
      for k in range(L):
        @pl.when(fvec[k] != 0)
        def _(h=h, k=k, splats=splats):
          sv = jnp.full((L,), splats[k])
          row = h * L + k

          def _fill_j(j, _):
            for u in range(8):
              rb[row, j, pl.ds(u * L, L)] = sv
            return 0
          lax.fori_loop(0, D // 128, _fill_j, 0)

  rbs = (rb0, rb1, rb2, rb3)
  sgs = (sem_g0, sem_g1, sem_g2, sem_g3)
  sss = (sem_s0, sem_s1, sem_s2, sem_s3)

  issue_gather(0, rb0, sem_g0)
  issue_gather(1, rb1, sem_g1)

  def _chunk(i, _):
    for s in range(NSLOT):
      @pl.when(i % NSLOT == s)
      def _(s=s):
        drain_gather(rbs[s], sgs[s])
        fill(i, rbs[s])
        issue_store(i, rbs[s], sss[s])
    for s in range(NSLOT):
      @pl.when(((i + 2) % NSLOT == s) & (i + 2 < NCHUNK))
      def _(s=s):
        @pl.when(i >= 2)
        def _():
          drain_store(rbs[s], sss[s])
        issue_gather(i + 2, rbs[s], sgs[s])
    return 0
  lax.fori_loop(0, NCHUNK, _chunk, 0)

  drain_store(rb2, sem_s2)
  drain_store(rb3, sem_s3)


_mesh = plsc.VectorSubcoreMesh(core_axis_name="c", subcore_axis_name="s")

_sc_call = functools.partial(
    pl.kernel,
    out_type=jax.ShapeDtypeStruct((T, D // 128, 128), jnp.float32),
    mesh=_mesh,
    compiler_params=pltpu.CompilerParams(needs_layout_passes=False),
    scratch_types=[
        pltpu.VMEM((T,), jnp.int32),
        pltpu.VMEM((TPW,), jnp.int32),
        pltpu.VMEM((TPW,), jnp.int32),
        pltpu.VMEM((TPW,), jnp.int32),
        pltpu.VMEM((K, D // 128, 128), jnp.float32),
        pltpu.VMEM((K, D // 128, 128), jnp.float32),
        pltpu.VMEM((K, D // 128, 128), jnp.float32),
        pltpu.VMEM((K, D // 128, 128), jnp.float32),
        pltpu.SemaphoreType.DMA,
        pltpu.SemaphoreType.DMA,
        pltpu.SemaphoreType.DMA,
        pltpu.SemaphoreType.DMA,
        pltpu.SemaphoreType.DMA,
        pltpu.SemaphoreType.DMA,
        pltpu.SemaphoreType.DMA,
        pltpu.SemaphoreType.DMA,
    ],
)(_body)


@jax.jit
def kernel(inputs_embeds, images_seq_mask, stacked_image_feats):
  del inputs_embeds
  mask_i = images_seq_mask.reshape(-1).astype(jnp.int32)
  src3 = stacked_image_feats.reshape(N_IMG, D // 128, 128)
  out = _sc_call(mask_i, src3)
  return out.reshape(-1)


# speedup vs baseline: 5925.4619x; 1.0274x over previous
import functools

import jax
import jax.numpy as jnp
from jax import lax
from jax.experimental import pallas as pl
from jax.experimental.pallas import tpu as pltpu
from jax.experimental.pallas import tpu_sc as plsc

B, S, D = 4, 4096, 1024
T = B * S
N_IMG = 8192
MAXI = N_IMG * D - 1
NC, NS, L = 2, 16, 16
NW = NC * NS
TPW = T // NW
K = 32
NCHUNK = TPW // K
NSLOT = 3
NV = TPW // L


def _body(mask_hbm, src_hbm, out_hbm,
          mask_v, grow_v, lane_v, flag_v,
          rb0, rb1, rb2,
          sem_g0, sem_g1, sem_g2,
          sem_s0, sem_s1, sem_s2):
  cid = lax.axis_index("c")
  sid = lax.axis_index("s")
  wid = sid * NC + cid
  t0 = wid * TPW

  pltpu.sync_copy(mask_hbm, mask_v)

  def _red(j, acc):
    a0, a1 = acc
    for u in range(0, 8, 2):
      a0 = a0 + mask_v[pl.ds((j * 8 + u) * L, L)]
      a1 = a1 + mask_v[pl.ds((j * 8 + u + 1) * L, L)]
    return (a0, a1)
  z = jnp.zeros((L,), jnp.int32)
  acc0, acc1 = lax.fori_loop(0, wid * (NV // 8), _red, (z, z))
  base0 = jnp.sum(acc0 + acc1)

  iota = lax.iota(jnp.int32, L)

  def _scan(j, carry):
    v = mask_v[pl.ds(t0 + j * L, L)]
    incl = jnp.cumsum(v)
    c = carry + incl - v
    is_row = (v > 0) & (c < N_IMG)
    e = jnp.clip(c * D - (1 - v), 0, MAXI)
    grow_v[pl.ds(j * L, L)] = e >> 10
    lane_v[pl.ds(j * L, L)] = e & (D - 1)
    flag_v[pl.ds(j * L, L)] = 1 - is_row.astype(jnp.int32)
    return carry + jnp.sum(v)
  lax.fori_loop(0, NV, _scan, base0)

  def issue_gather(i, rb, sem):
    pltpu.async_copy(src_hbm.at[grow_v.at[pl.ds(i * K, K)]], rb, sem)

  def drain_gather(rb, sem):
    pltpu.make_async_copy(src_hbm.at[pl.ds(0, K)], rb, sem).wait()

  def issue_store(i, rb, sem):
    pltpu.async_copy(rb, out_hbm.at[pl.ds(t0 + i * K, K)], sem)

  def drain_store(rb, sem):
    pltpu.make_async_copy(rb, out_hbm.at[pl.ds(0, K)], sem).wait()

  def fill(i, rb):
    for h in range(K // L):
      off = i * K + h * L
      fvec = flag_v[pl.ds(off, L)]
      lvec = lane_v[pl.ds(off, L)]
      splats = plsc.load_gather(rb, [h * L + iota, lvec >> 7, lvec & 127])
      for k in range(L):
        @pl.when(fvec[k] != 0)
        def _(h=h, k=k, splats=splats):
          sv = jnp.full((L,), splats[k])
          row = h * L + k

          def _fill_j(j, _):
            for u in range(8):
              rb[row, j, pl.ds(u * L, L)] = sv
            return 0
          lax.fori_loop(0, D // 128, _fill_j, 0)

  rbs = (rb0, rb1, rb2)
  sgs = (sem_g0, sem_g1, sem_g2)
  sss = (sem_s0, sem_s1, sem_s2)

  issue_gather(0, rb0, sem_g0)
  issue_gather(1, rb1, sem_g1)

  def _chunk(i, _):
    for s in range(NSLOT):
      @pl.when(i % NSLOT == s)
      def _(s=s):
        drain_gather(rbs[s], sgs[s])
        fill(i, rbs[s])
        issue_store(i, rbs[s], sss[s])
    for s in range(NSLOT):
      @pl.when(((i + 2) % NSLOT == s) & (i + 2 < NCHUNK))
      def _(s=s):
        @pl.when(i >= 2)
        def _():
          drain_store(rbs[s], sss[s])
        issue_gather(i + 2, rbs[s], sgs[s])
    return 0
  lax.fori_loop(0, NCHUNK, _chunk, 0)

  drain_store(rbs[(NCHUNK - 2) % NSLOT], sss[(NCHUNK - 2) % NSLOT])
  drain_store(rbs[(NCHUNK - 1) % NSLOT], sss[(NCHUNK - 1) % NSLOT])


_mesh = plsc.VectorSubcoreMesh(core_axis_name="c", subcore_axis_name="s")

_sc_call = functools.partial(
    pl.kernel,
    out_type=jax.ShapeDtypeStruct((T, D // 128, 128), jnp.float32),
    mesh=_mesh,
    compiler_params=pltpu.CompilerParams(needs_layout_passes=False),
    scratch_types=[
        pltpu.VMEM((T,), jnp.int32),
        pltpu.VMEM((TPW,), jnp.int32),
        pltpu.VMEM((TPW,), jnp.int32),
        pltpu.VMEM((TPW,), jnp.int32),
        pltpu.VMEM((K, D // 128, 128), jnp.float32),
        pltpu.VMEM((K, D // 128, 128), jnp.float32),
        pltpu.VMEM((K, D // 128, 128), jnp.float32),
        pltpu.SemaphoreType.DMA,
        pltpu.SemaphoreType.DMA,
        pltpu.SemaphoreType.DMA,
        pltpu.SemaphoreType.DMA,
        pltpu.SemaphoreType.DMA,
        pltpu.SemaphoreType.DMA,
    ],
)(_body)


@jax.jit
def kernel(inputs_embeds, images_seq_mask, stacked_image_feats):
  del inputs_embeds
  mask_i = images_seq_mask.reshape(-1).astype(jnp.int32)
  src3 = stacked_image_feats.reshape(N_IMG, D // 128, 128)
  out = _sc_call(mask_i, src3)
  return out.reshape(-1)
